# baseline (speedup 1.0000x reference)
import functools

import jax
import jax.numpy as jnp
from jax import lax
from jax.experimental import pallas as pl
from jax.experimental.pallas import tpu as pltpu

D_MODEL = 1024
NUM_EXPERTS = 8
TOP_K = 2
BATCH = 4
SEQ_LEN = 2048
D_FF = 4 * D_MODEL
T = BATCH * SEQ_LEN
P = TOP_K * T
BLOCK = 512
CAP = P + NUM_EXPERTS * BLOCK
NB = CAP // BLOCK
NBP = 48
FF_T = 512
NF = D_FF // FF_T


_A_TOK = 1024


def _router_body(x_ref, wr_ref, e0_ref, e1_ref, w0_ref, w1_ref):
    xb = x_ref[...]
    wr = wr_ref[...]
    logits = lax.dot_general(wr, xb, (((1,), (1,)), ((), ())),
                             preferred_element_type=jnp.float32)
    sub = lax.broadcasted_iota(jnp.int32, (NUM_EXPERTS, _A_TOK), 0)
    m0 = jnp.max(logits, axis=0, keepdims=True)
    i0 = jnp.min(jnp.where(logits == m0, sub, NUM_EXPERTS), axis=0, keepdims=True)
    l2 = jnp.where(sub == i0, -jnp.inf, logits)
    m1 = jnp.max(l2, axis=0, keepdims=True)
    i1 = jnp.min(jnp.where(l2 == m1, sub, NUM_EXPERTS), axis=0, keepdims=True)
    p = jnp.exp(m1 - m0)
    w0 = 1.0 / (1.0 + p)
    e0_ref[...] = i0[None]
    e1_ref[...] = i1[None]
    w0_ref[...] = w0[None]
    w1_ref[...] = (p * w0)[None]


def _router(x_flat, W_router):
    n = T // _A_TOK
    out = jax.ShapeDtypeStruct((n, 1, _A_TOK), jnp.int32)
    outf = jax.ShapeDtypeStruct((n, 1, _A_TOK), jnp.float32)
    e0, e1, w0, w1 = pl.pallas_call(
        _router_body,
        grid=(n,),
        in_specs=[
            pl.BlockSpec((_A_TOK, D_MODEL), lambda i: (i, 0)),
            pl.BlockSpec((NUM_EXPERTS, D_MODEL), lambda i: (0, 0)),
        ],
        out_specs=[
            pl.BlockSpec((1, 1, _A_TOK), lambda i: (i, 0, 0)),
            pl.BlockSpec((1, 1, _A_TOK), lambda i: (i, 0, 0)),
            pl.BlockSpec((1, 1, _A_TOK), lambda i: (i, 0, 0)),
            pl.BlockSpec((1, 1, _A_TOK), lambda i: (i, 0, 0)),
        ],
        out_shape=[out, out, outf, outf],
    )(x_flat, W_router)
    return (e0.reshape(T), e1.reshape(T), w0.reshape(T), w1.reshape(T))


def _route(e0, e1, w0, w1):
    e = jnp.concatenate([e0, e1])
    w = jnp.concatenate([w0, w1])
    cnt = jnp.bincount(e, length=NUM_EXPERTS)
    padded = (cnt + BLOCK - 1) // BLOCK * BLOCK
    ends = jnp.cumsum(padded)
    starts = ends - padded
    cstart = jnp.cumsum(cnt) - cnt
    order = jnp.argsort(e, stable=True)
    e_sorted = e[order]
    dst_sorted = starts[e_sorted] + jnp.arange(P) - cstart[e_sorted]
    dst = jnp.zeros((P,), jnp.int32).at[order].set(dst_sorted.astype(jnp.int32))
    row_token = jnp.zeros((CAP,), jnp.int32).at[dst].set(
        (jnp.arange(P) % T).astype(jnp.int32))
    gates = jnp.zeros((CAP,), jnp.float32).at[dst].set(w)
    bs = jnp.arange(NBP) * BLOCK
    blk_e = jnp.clip(jnp.searchsorted(ends, bs, side="right"), 0, NUM_EXPERTS - 1)
    blk_a = (bs < ends[-1]).astype(jnp.int32)
    return (row_token, gates, dst[:T], dst[T:],
            blk_e.astype(jnp.int32), blk_a)


def _ffn_body(be_ref, ba_ref, x_ref, w1_ref, w2_ref, b1_ref, b2_ref, g_ref,
              y_ref):
    b = pl.program_id(0)
    f = pl.program_id(1)
    e = be_ref[b]

    @pl.when(ba_ref[b] != 0)
    def _():
        xb = x_ref[...]
        w1 = w1_ref[0]
        h = lax.dot_general(xb, w1, (((1,), (1,)), ((), ())),
                            preferred_element_type=jnp.float32)
        h = jnp.maximum(h + b1_ref[e, f][None, :], 0.0)
        w2 = w2_ref[0]
        part = lax.dot_general(h.astype(jnp.bfloat16), w2,
                               (((1,), (1,)), ((), ())),
                               preferred_element_type=jnp.float32)

        @pl.when(f == 0)
        def _():
            y_ref[...] = part + b2_ref[e][None, :]

        @pl.when(f > 0)
        def _():
            y_ref[...] += part

        @pl.when(f == NF - 1)
        def _():
            y_ref[...] *= g_ref[0]


def _ffn(blk_e, blk_a, xbuf, gates, W1, b1, W2, b2):
    b1r = b1.reshape(NUM_EXPERTS, NF, FF_T)
    g3 = gates.reshape(NB, BLOCK, 1)
    grid_spec = pltpu.PrefetchScalarGridSpec(
        num_scalar_prefetch=2,
        grid=(NB, NF),
        in_specs=[
            pl.BlockSpec((BLOCK, D_MODEL), lambda b, f, be, ba: (b, 0)),
            pl.BlockSpec((1, FF_T, D_MODEL),
                         lambda b, f, be, ba: (be[b], jnp.where(ba[b] != 0, f, 0), 0)),
            pl.BlockSpec((1, D_MODEL, FF_T),
                         lambda b, f, be, ba: (be[b], 0, jnp.where(ba[b] != 0, f, 0))),
            pl.BlockSpec((NUM_EXPERTS, NF, FF_T), lambda b, f, be, ba: (0, 0, 0)),
            pl.BlockSpec((NUM_EXPERTS, D_MODEL), lambda b, f, be, ba: (0, 0)),
            pl.BlockSpec((1, BLOCK, 1), lambda b, f, be, ba: (b, 0, 0)),
        ],
        out_specs=pl.BlockSpec((BLOCK, D_MODEL), lambda b, f, be, ba: (b, 0)),
    )
    return pl.pallas_call(
        _ffn_body,
        grid_spec=grid_spec,
        out_shape=jax.ShapeDtypeStruct((CAP, D_MODEL), jnp.float32),
        compiler_params=pltpu.CompilerParams(
            dimension_semantics=("arbitrary", "arbitrary")),
    )(blk_e, blk_a, xbuf, W1, W2, b1r, b2, g3)


def kernel(x, W_router, W1, b1, W2, b2):
    x_flat = x.reshape(T, D_MODEL)
    e0, e1, w0, w1 = _router(x_flat, W_router)
    row_token, gates, dst0, dst1, blk_e, blk_a = _route(e0, e1, w0, w1)
    x_bf = x_flat.astype(jnp.bfloat16)
    xbuf = x_bf[row_token]
    y = _ffn(blk_e, blk_a, xbuf, gates,
             W1.astype(jnp.bfloat16), b1, W2.astype(jnp.bfloat16), b2)
    out = y[dst0] + y[dst1]
    return out.reshape(BATCH, SEQ_LEN, D_MODEL)


# speedup vs baseline: 1.2015x; 1.2015x over previous
import functools

import jax
import jax.numpy as jnp
from jax import lax
from jax.experimental import pallas as pl
from jax.experimental.pallas import tpu as pltpu

D_MODEL = 1024
NUM_EXPERTS = 8
TOP_K = 2
BATCH = 4
SEQ_LEN = 2048
D_FF = 4 * D_MODEL
T = BATCH * SEQ_LEN
P = TOP_K * T
BLOCK = 512
CAP = P + NUM_EXPERTS * BLOCK
NB = CAP // BLOCK
NBP = 48
FF_T = 512
NF = D_FF // FF_T


_A_TOK = 1024


def _router_body(x_ref, wr_ref, e0_ref, e1_ref, w0_ref, w1_ref):
    xb = x_ref[...]
    wr = wr_ref[...]
    logits = lax.dot_general(wr, xb, (((1,), (1,)), ((), ())),
                             preferred_element_type=jnp.float32)
    sub = lax.broadcasted_iota(jnp.int32, (NUM_EXPERTS, _A_TOK), 0)
    m0 = jnp.max(logits, axis=0, keepdims=True)
    i0 = jnp.min(jnp.where(logits == m0, sub, NUM_EXPERTS), axis=0, keepdims=True)
    l2 = jnp.where(sub == i0, -jnp.inf, logits)
    m1 = jnp.max(l2, axis=0, keepdims=True)
    i1 = jnp.min(jnp.where(l2 == m1, sub, NUM_EXPERTS), axis=0, keepdims=True)
    p = jnp.exp(m1 - m0)
    w0 = 1.0 / (1.0 + p)
    e0_ref[...] = i0[None]
    e1_ref[...] = i1[None]
    w0_ref[...] = w0[None]
    w1_ref[...] = (p * w0)[None]


def _router(x_flat, W_router):
    n = T // _A_TOK
    out = jax.ShapeDtypeStruct((n, 1, _A_TOK), jnp.int32)
    outf = jax.ShapeDtypeStruct((n, 1, _A_TOK), jnp.float32)
    e0, e1, w0, w1 = pl.pallas_call(
        _router_body,
        grid=(n,),
        in_specs=[
            pl.BlockSpec((_A_TOK, D_MODEL), lambda i: (i, 0)),
            pl.BlockSpec((NUM_EXPERTS, D_MODEL), lambda i: (0, 0)),
        ],
        out_specs=[
            pl.BlockSpec((1, 1, _A_TOK), lambda i: (i, 0, 0)),
            pl.BlockSpec((1, 1, _A_TOK), lambda i: (i, 0, 0)),
            pl.BlockSpec((1, 1, _A_TOK), lambda i: (i, 0, 0)),
            pl.BlockSpec((1, 1, _A_TOK), lambda i: (i, 0, 0)),
        ],
        out_shape=[out, out, outf, outf],
    )(x_flat, W_router)
    return (e0.reshape(T), e1.reshape(T), w0.reshape(T), w1.reshape(T))


def _route(e0, e1, w0, w1):
    e = jnp.concatenate([e0, e1])
    w = jnp.concatenate([w0, w1])
    cnt = jnp.bincount(e, length=NUM_EXPERTS)
    padded = (cnt + BLOCK - 1) // BLOCK * BLOCK
    ends = jnp.cumsum(padded)
    starts = ends - padded
    cstart = jnp.cumsum(cnt) - cnt
    order = jnp.argsort(e, stable=True)
    e_sorted = e[order]
    dst_sorted = starts[e_sorted] + jnp.arange(P) - cstart[e_sorted]
    dst = jnp.zeros((P,), jnp.int32).at[order].set(dst_sorted.astype(jnp.int32))
    row_token = jnp.zeros((CAP,), jnp.int32).at[dst].set(
        (jnp.arange(P) % T).astype(jnp.int32))
    gates = jnp.zeros((CAP,), jnp.float32).at[dst].set(w)
    bs = jnp.arange(NBP) * BLOCK
    blk_e = jnp.clip(jnp.searchsorted(ends, bs, side="right"), 0, NUM_EXPERTS - 1)
    blk_a = (bs < ends[-1]).astype(jnp.int32)
    return (row_token, gates, dst[:T], dst[T:],
            blk_e.astype(jnp.int32), blk_a)


def _ffn_body(be_ref, ba_ref, x_ref, w1_ref, w2_ref, b1_ref, b2_ref, g_ref,
              y_ref):
    b = pl.program_id(0)
    e = be_ref[b]

    @pl.when(ba_ref[b] != 0)
    def _():
        xb = x_ref[...]
        w1 = w1_ref[0]
        h = lax.dot_general(xb, w1, (((1,), (1,)), ((), ())),
                            preferred_element_type=jnp.float32)
        h = jnp.maximum(h + b1_ref[e][None, :], 0.0)
        w2 = w2_ref[0]
        part = lax.dot_general(h.astype(jnp.bfloat16), w2,
                               (((1,), (1,)), ((), ())),
                               preferred_element_type=jnp.float32)
        y_ref[...] = (part + b2_ref[e][None, :]) * g_ref[0]


def _ffn(blk_e, blk_a, xbuf, gates, W1, b1, W2, b2):
    g3 = gates.reshape(NB, BLOCK, 1)
    grid_spec = pltpu.PrefetchScalarGridSpec(
        num_scalar_prefetch=2,
        grid=(NB,),
        in_specs=[
            pl.BlockSpec((BLOCK, D_MODEL), lambda b, be, ba: (b, 0)),
            pl.BlockSpec((1, D_FF, D_MODEL), lambda b, be, ba: (be[b], 0, 0)),
            pl.BlockSpec((1, D_MODEL, D_FF), lambda b, be, ba: (be[b], 0, 0)),
            pl.BlockSpec((NUM_EXPERTS, D_FF), lambda b, be, ba: (0, 0)),
            pl.BlockSpec((NUM_EXPERTS, D_MODEL), lambda b, be, ba: (0, 0)),
            pl.BlockSpec((1, BLOCK, 1), lambda b, be, ba: (b, 0, 0)),
        ],
        out_specs=pl.BlockSpec((BLOCK, D_MODEL), lambda b, be, ba: (b, 0)),
    )
    return pl.pallas_call(
        _ffn_body,
        grid_spec=grid_spec,
        out_shape=jax.ShapeDtypeStruct((CAP, D_MODEL), jnp.float32),
        compiler_params=pltpu.CompilerParams(
            dimension_semantics=("arbitrary",)),
    )(blk_e, blk_a, xbuf, W1, W2, b1, b2, g3)


def kernel(x, W_router, W1, b1, W2, b2):
    x_flat = x.reshape(T, D_MODEL)
    e0, e1, w0, w1 = _router(x_flat, W_router)
    row_token, gates, dst0, dst1, blk_e, blk_a = _route(e0, e1, w0, w1)
    x_bf = x_flat.astype(jnp.bfloat16)
    xbuf = x_bf[row_token]
    y = _ffn(blk_e, blk_a, xbuf, gates,
             W1.astype(jnp.bfloat16), b1, W2.astype(jnp.bfloat16), b2)
    out = y[dst0] + y[dst1]
    return out.reshape(BATCH, SEQ_LEN, D_MODEL)
